# baseline (speedup 1.0000x reference)
import functools
import math

import jax
import jax.numpy as jnp
from jax import lax
from jax.experimental import pallas as pl
from jax.experimental.pallas import tpu as pltpu
from jax.experimental.pallas import tpu_sc as plsc

_NC = 2
_NS = 16
_NW = _NC * _NS
_LANES = 16
_CHUNK = 128
_NBUF = 4


@functools.lru_cache(maxsize=None)
def _make_lookup(B, V, D, scale):
    assert B % (8 * _NW) == 0
    assert D % _LANES == 0
    b_per_w = B // _NW
    assert b_per_w % (_CHUNK * _NBUF) == 0
    n_chunks = b_per_w // _CHUNK
    n_outer = n_chunks // _NBUF
    mesh = plsc.VectorSubcoreMesh(core_axis_name="c", subcore_axis_name="s")

    @functools.partial(
        pl.kernel,
        mesh=mesh,
        out_type=jax.ShapeDtypeStruct((B, D), jnp.float32),
        scratch_types=(
            [pltpu.VMEM((b_per_w,), jnp.int32)]
            + [pltpu.VMEM((_CHUNK, D), jnp.float32) for _ in range(2 * _NBUF)]
            + [pltpu.SemaphoreType.DMA for _ in range(2 * _NBUF)]
        ),
        compiler_params=pltpu.CompilerParams(use_tc_tiling_on_sc=False),
    )
    def lookup(idx_hbm, table_hbm, out_hbm, idx_v, *rest):
        g_buf = rest[:_NBUF]
        o_buf = rest[_NBUF:2 * _NBUF]
        sem_g = rest[2 * _NBUF:3 * _NBUF]
        sem_o = rest[3 * _NBUF:]

        wid = lax.axis_index("s") * _NC + lax.axis_index("c")
        base = wid * b_per_w
        pltpu.sync_copy(idx_hbm.at[pl.ds(base, b_per_w)], idx_v)

        def start_gather(b, c):
            start = pl.multiple_of(c * _CHUNK, _CHUNK)
            pltpu.async_copy(
                table_hbm.at[idx_v.at[pl.ds(start, _CHUNK)]], g_buf[b], sem_g[b]
            )

        for b in range(_NBUF):
            start_gather(b, b)

        def outer(g, carry):
            for b in range(_NBUF):
                c = g * _NBUF + b
                pltpu.make_async_copy(
                    table_hbm.at[pl.ds(0, _CHUNK)], g_buf[b], sem_g[b]
                ).wait()

                def row_body(r, acc, b=b):
                    for j in range(D // _LANES):
                        sl = g_buf[b][r, pl.ds(j * _LANES, _LANES)]
                        o_buf[b][r, pl.ds(j * _LANES, _LANES)] = sl * scale
                    return acc

                lax.fori_loop(0, _CHUNK, row_body, 0, unroll=4)

                @pl.when(g > 0)
                def _(b=b):
                    pltpu.make_async_copy(
                        o_buf[b], out_hbm.at[pl.ds(0, _CHUNK)], sem_o[b]
                    ).wait()

                start = pl.multiple_of(c * _CHUNK, _CHUNK)
                pltpu.async_copy(
                    o_buf[b], out_hbm.at[pl.ds(base + start, _CHUNK)], sem_o[b]
                )

                @pl.when(g < n_outer - 1)
                def _(b=b, c=c):
                    start_gather(b, c + _NBUF)
            return carry

        lax.fori_loop(0, n_outer, outer, 0)

        for b in range(_NBUF):
            pltpu.make_async_copy(
                o_buf[b], out_hbm.at[pl.ds(0, _CHUNK)], sem_o[b]
            ).wait()

    return lookup


def kernel(tokens, table):
    n, t = tokens.shape
    V, D = table.shape
    B = n * t
    idx = tokens.reshape(B).astype(jnp.int32)
    out = _make_lookup(B, V, D, float(math.sqrt(D)))(idx, table)
    return out.reshape(n, t, D)


# speedup vs baseline: 1.0281x; 1.0281x over previous
import functools
import math

import jax
import jax.numpy as jnp
from jax import lax
from jax.experimental import pallas as pl
from jax.experimental.pallas import tpu as pltpu
from jax.experimental.pallas import tpu_sc as plsc

_NC = 2
_NS = 16
_NW = _NC * _NS
_CHUNK = 128
_NRING = 8
_DEPTH = 4


@functools.lru_cache(maxsize=None)
def _make_lookup(B, V, D):
    assert B % (8 * _NW) == 0
    b_per_w = B // _NW
    assert b_per_w % (_CHUNK * _NRING) == 0
    n_chunks = b_per_w // _CHUNK
    n_outer = n_chunks // _NRING
    mesh = plsc.VectorSubcoreMesh(core_axis_name="c", subcore_axis_name="s")

    @functools.partial(
        pl.kernel,
        mesh=mesh,
        out_type=jax.ShapeDtypeStruct((B, D), jnp.float32),
        scratch_types=(
            [pltpu.VMEM((b_per_w,), jnp.int32)]
            + [pltpu.VMEM((_CHUNK, D), jnp.float32) for _ in range(_NRING)]
            + [pltpu.SemaphoreType.DMA for _ in range(2 * _NRING)]
        ),
        compiler_params=pltpu.CompilerParams(use_tc_tiling_on_sc=False),
    )
    def lookup(idx_hbm, table_hbm, out_hbm, idx_v, *rest):
        buf = rest[:_NRING]
        sem_g = rest[_NRING:2 * _NRING]
        sem_o = rest[2 * _NRING:]

        wid = lax.axis_index("s") * _NC + lax.axis_index("c")
        base = wid * b_per_w
        pltpu.sync_copy(idx_hbm.at[pl.ds(base, b_per_w)], idx_v)

        def start_gather(b, c):
            start = pl.multiple_of(c * _CHUNK, _CHUNK)
            pltpu.async_copy(
                table_hbm.at[idx_v.at[pl.ds(start, _CHUNK)]], buf[b], sem_g[b]
            )

        for b in range(_DEPTH):
            start_gather(b, b)

        def outer(g, carry):
            for b in range(_NRING):
                c = g * _NRING + b
                pltpu.make_async_copy(
                    table_hbm.at[pl.ds(0, _CHUNK)], buf[b], sem_g[b]
                ).wait()
                start = pl.multiple_of(c * _CHUNK, _CHUNK)
                pltpu.async_copy(
                    buf[b], out_hbm.at[pl.ds(base + start, _CHUNK)], sem_o[b]
                )

                bpf = (b + _DEPTH) % _NRING

                @pl.when(c + _DEPTH < n_chunks)
                def _(b=b, bpf=bpf, c=c):
                    @pl.when(c + _DEPTH >= _NRING)
                    def _():
                        pltpu.make_async_copy(
                            buf[bpf], out_hbm.at[pl.ds(0, _CHUNK)], sem_o[bpf]
                        ).wait()

                    start_gather(bpf, c + _DEPTH)
            return carry

        lax.fori_loop(0, n_outer, outer, 0)

        for b in range(_NRING):
            pltpu.make_async_copy(
                buf[b], out_hbm.at[pl.ds(0, _CHUNK)], sem_o[b]
            ).wait()

    return lookup


def kernel(tokens, table):
    n, t = tokens.shape
    V, D = table.shape
    B = n * t
    idx = tokens.T.reshape(B).astype(jnp.int32)
    table_scaled = table * jnp.float32(math.sqrt(D))
    out = _make_lookup(B, V, D)(idx, table_scaled)
    return out.reshape(t, n, D).transpose(1, 0, 2)
